# baseline (speedup 1.0000x reference)
import jax
import jax.numpy as jnp
from jax import lax
from jax.experimental import pallas as pl
from jax.experimental.pallas import tpu as pltpu
from jax.experimental.pallas import tpu_sc as plsc

NUM_VIEWS = 16
TOKEN_DIM = 256
BATCH = 16384
NUM_CORES = 2
NUM_SUBCORES = 16
NUM_WORKERS = NUM_CORES * NUM_SUBCORES
ROWS_PER_WORKER = BATCH // NUM_WORKERS
CHUNK = 128
NUM_CHUNKS = ROWS_PER_WORKER // CHUNK


def _gather_body(idx_hbm, table_hbm, out_hbm, idx_v, buf0, buf1, sem0, sem1):
    wid = lax.axis_index("s") * NUM_CORES + lax.axis_index("c")
    base = wid * ROWS_PER_WORKER
    pltpu.sync_copy(idx_hbm.at[wid], idx_v)
    bufs = (buf0, buf1)
    sems = (sem0, sem1)
    copies = [None, None]
    copies[0] = pltpu.async_copy(table_hbm.at[idx_v.at[0]], bufs[0], sems[0])
    for j in range(NUM_CHUNKS):
        cur = j % 2
        nxt = (j + 1) % 2
        if j + 1 < NUM_CHUNKS:
            copies[nxt] = pltpu.async_copy(
                table_hbm.at[idx_v.at[j + 1]], bufs[nxt], sems[nxt])
        copies[cur].wait()
        pltpu.sync_copy(bufs[cur], out_hbm.at[pl.ds(base + j * CHUNK, CHUNK)])


@jax.jit
def kernel(view_id, view_embed):
    idx = view_id.astype(jnp.int32).reshape(NUM_WORKERS, NUM_CHUNKS, CHUNK)
    run = pl.kernel(
        _gather_body,
        out_type=jax.ShapeDtypeStruct((BATCH, TOKEN_DIM), jnp.float32),
        mesh=plsc.VectorSubcoreMesh(core_axis_name="c", subcore_axis_name="s"),
        scratch_types=[
            pltpu.VMEM((NUM_CHUNKS, CHUNK), jnp.int32),
            pltpu.VMEM((CHUNK, TOKEN_DIM), jnp.float32),
            pltpu.VMEM((CHUNK, TOKEN_DIM), jnp.float32),
            pltpu.SemaphoreType.DMA,
            pltpu.SemaphoreType.DMA,
        ],
    )
    return run(idx, view_embed)


# speedup vs baseline: 1.0315x; 1.0315x over previous
import jax
import jax.numpy as jnp
from jax import lax
from jax.experimental import pallas as pl
from jax.experimental.pallas import tpu as pltpu
from jax.experimental.pallas import tpu_sc as plsc

NUM_VIEWS = 16
TOKEN_DIM = 256
BATCH = 16384
NUM_CORES = 2
NUM_SUBCORES = 16
NUM_WORKERS = NUM_CORES * NUM_SUBCORES
ROWS_PER_WORKER = BATCH // NUM_WORKERS
CHUNK = 128
NUM_CHUNKS = ROWS_PER_WORKER // CHUNK
NBUF = 3


def _gather_body(idx_hbm, table_hbm, out_hbm, idx_v,
                 buf0, buf1, buf2, gs0, gs1, gs2, ws0, ws1, ws2, ws3):
    wid = lax.axis_index("s") * NUM_CORES + lax.axis_index("c")
    base = wid * ROWS_PER_WORKER

    pltpu.sync_copy(idx_hbm.at[wid], idx_v)

    bufs = (buf0, buf1, buf2)
    gsems = (gs0, gs1, gs2)
    wsems = (ws0, ws1, ws2, ws3)
    gathers = [None] * NUM_CHUNKS
    writes = [None] * NUM_CHUNKS

    def start_gather(j):
        gathers[j] = pltpu.async_copy(
            table_hbm.at[idx_v.at[j]], bufs[j % NBUF], gsems[j % NBUF])

    for j in range(min(NBUF, NUM_CHUNKS)):
        start_gather(j)
    for j in range(NUM_CHUNKS):
        gathers[j].wait()
        writes[j] = pltpu.async_copy(
            bufs[j % NBUF], out_hbm.at[pl.ds(base + j * CHUNK, CHUNK)],
            wsems[j])
        nxt = j + NBUF
        if nxt < NUM_CHUNKS:
            writes[nxt - NBUF].wait()
            start_gather(nxt)
    for j in range(max(0, NUM_CHUNKS - NBUF), NUM_CHUNKS):
        writes[j].wait()


@jax.jit
def kernel(view_id, view_embed):
    idx = view_id.astype(jnp.int32).reshape(NUM_WORKERS, NUM_CHUNKS, CHUNK)
    run = pl.kernel(
        _gather_body,
        out_type=jax.ShapeDtypeStruct((BATCH, TOKEN_DIM), jnp.float32),
        mesh=plsc.VectorSubcoreMesh(core_axis_name="c", subcore_axis_name="s"),
        scratch_types=[
            pltpu.VMEM((NUM_CHUNKS, CHUNK), jnp.int32),
            pltpu.VMEM((CHUNK, TOKEN_DIM), jnp.float32),
            pltpu.VMEM((CHUNK, TOKEN_DIM), jnp.float32),
            pltpu.VMEM((CHUNK, TOKEN_DIM), jnp.float32),
            pltpu.SemaphoreType.DMA,
            pltpu.SemaphoreType.DMA,
            pltpu.SemaphoreType.DMA,
            pltpu.SemaphoreType.DMA,
            pltpu.SemaphoreType.DMA,
            pltpu.SemaphoreType.DMA,
            pltpu.SemaphoreType.DMA,
        ],
    )
    return run(idx, view_embed)
